# initial kernel scaffold (unmeasured)
import jax
import jax.numpy as jnp
from jax import lax
from jax.experimental import pallas as pl
from jax.experimental.pallas import tpu as pltpu

N_DEV = 8


def kernel(x, w_mat, scale_x, scale_w):
    M, K_shard = x.shape
    _, N = w_mat.shape
    B = M // N_DEV

    x8 = x.astype(jnp.float8_e5m2)
    w8 = w_mat.astype(jnp.float8_e5m2)

    def body(x_ref, w_ref, sx_ref, sw_ref, out_ref,
             xg_ref, wbuf_ref, x_send, x_recv, w_send, w_recv):
        me = lax.axis_index("i")
        right = lax.rem(me + 1, N_DEV)
        left = lax.rem(me + N_DEV - 1, N_DEV)

        bar = pltpu.get_barrier_semaphore()
        for off in range(1, N_DEV):
            peer = lax.rem(me + off, N_DEV)
            pl.semaphore_signal(bar, inc=1, device_id=(peer,),
                                device_id_type=pl.DeviceIdType.MESH)
        pl.semaphore_wait(bar, N_DEV - 1)

        x_rdmas = []
        for off in range(1, N_DEV):
            dst = lax.rem(me + off, N_DEV)
            r = pltpu.make_async_remote_copy(
                src_ref=x_ref.at[pl.ds(dst * B, B), :],
                dst_ref=xg_ref.at[off],
                send_sem=x_send.at[off],
                recv_sem=x_recv.at[off],
                device_id=(dst,),
                device_id_type=pl.DeviceIdType.MESH,
            )
            r.start()
            x_rdmas.append(r)

        xg_ref[0] = x_ref[pl.ds(me * B, B), :]

        w_rdmas = []
        r0 = pltpu.make_async_remote_copy(
            src_ref=w_ref,
            dst_ref=wbuf_ref.at[0],
            send_sem=w_send.at[0],
            recv_sem=w_recv.at[0],
            device_id=(right,),
            device_id_type=pl.DeviceIdType.MESH,
        )
        r0.start()
        w_rdmas.append(r0)

        out_ref[...] = lax.dot_general(
            xg_ref[0], w_ref[...],
            (((1,), (0,)), ((), ())),
            preferred_element_type=jnp.float32,
        )

        for h in range(N_DEV - 1):
            pltpu.make_async_remote_copy(
                src_ref=wbuf_ref.at[h],
                dst_ref=wbuf_ref.at[h],
                send_sem=w_send.at[h],
                recv_sem=w_recv.at[h],
                device_id=(left,),
                device_id_type=pl.DeviceIdType.MESH,
            ).wait_recv()

            if h + 1 < N_DEV - 1:
                r = pltpu.make_async_remote_copy(
                    src_ref=wbuf_ref.at[h],
                    dst_ref=wbuf_ref.at[h + 1],
                    send_sem=w_send.at[h + 1],
                    recv_sem=w_recv.at[h + 1],
                    device_id=(right,),
                    device_id_type=pl.DeviceIdType.MESH,
                )
                r.start()
                w_rdmas.append(r)

            pltpu.make_async_remote_copy(
                src_ref=xg_ref.at[h + 1],
                dst_ref=xg_ref.at[h + 1],
                send_sem=x_recv.at[h + 1],
                recv_sem=x_recv.at[h + 1],
                device_id=(left,),
                device_id_type=pl.DeviceIdType.MESH,
            ).wait_recv()

            out_ref[...] = out_ref[...] + lax.dot_general(
                xg_ref[h + 1], wbuf_ref[h],
                (((1,), (0,)), ((), ())),
                preferred_element_type=jnp.float32,
            )

        s = sx_ref[0] * sw_ref[0]
        out_ref[...] = jnp.maximum(out_ref[...] * s, 0.0)

        for r in x_rdmas + w_rdmas:
            r.wait_send()

    return pl.pallas_call(
        body,
        out_shape=jax.ShapeDtypeStruct((B, N), jnp.float32),
        in_specs=[
            pl.BlockSpec(memory_space=pltpu.VMEM),
            pl.BlockSpec(memory_space=pltpu.VMEM),
            pl.BlockSpec(memory_space=pltpu.SMEM),
            pl.BlockSpec(memory_space=pltpu.SMEM),
        ],
        out_specs=pl.BlockSpec(memory_space=pltpu.VMEM),
        scratch_shapes=[
            pltpu.VMEM((N_DEV, B, K_shard), jnp.float8_e5m2),
            pltpu.VMEM((N_DEV - 1, K_shard, N), jnp.float8_e5m2),
            pltpu.SemaphoreType.DMA((N_DEV,)),
            pltpu.SemaphoreType.DMA((N_DEV,)),
            pltpu.SemaphoreType.DMA((N_DEV - 1,)),
            pltpu.SemaphoreType.DMA((N_DEV - 1,)),
        ],
        compiler_params=pltpu.CompilerParams(collective_id=0),
    )(x8, w8, scale_x, scale_w)


# baseline (device time: 382066 ns/iter reference)
import jax
import jax.numpy as jnp
from jax import lax
from jax.experimental import pallas as pl
from jax.experimental.pallas import tpu as pltpu

N_DEV = 8


def kernel(x, w_mat, scale_x, scale_w):
    M, K_shard = x.shape
    _, N = w_mat.shape
    B = M // N_DEV

    x8 = x.astype(jnp.float8_e5m2)
    w8 = w_mat.astype(jnp.float8_e5m2)

    def body(x_ref, w_ref, sx_ref, sw_ref, out_ref,
             xg_ref, wbuf_ref, x_send, x_recv, w_send, w_recv):
        me = lax.axis_index("i")
        right = lax.rem(me + 1, N_DEV)
        left = lax.rem(me + N_DEV - 1, N_DEV)

        bar = pltpu.get_barrier_semaphore()
        for off in range(1, N_DEV):
            peer = lax.rem(me + off, N_DEV)
            pl.semaphore_signal(bar, inc=1, device_id=(peer,),
                                device_id_type=pl.DeviceIdType.MESH)
        pl.semaphore_wait(bar, N_DEV - 1)

        x_rdmas = []
        for off in range(1, N_DEV):
            dst = lax.rem(me + off, N_DEV)
            r = pltpu.make_async_remote_copy(
                src_ref=x_ref.at[pl.ds(dst * B, B), :],
                dst_ref=xg_ref.at[off],
                send_sem=x_send.at[off],
                recv_sem=x_recv.at[off],
                device_id=(dst,),
                device_id_type=pl.DeviceIdType.MESH,
            )
            r.start()
            x_rdmas.append(r)

        xg_ref[0] = x_ref[pl.ds(me * B, B), :]

        w_rdmas = []
        r0 = pltpu.make_async_remote_copy(
            src_ref=w_ref,
            dst_ref=wbuf_ref.at[0],
            send_sem=w_send.at[0],
            recv_sem=w_recv.at[0],
            device_id=(right,),
            device_id_type=pl.DeviceIdType.MESH,
        )
        r0.start()
        w_rdmas.append(r0)

        out_ref[...] = lax.dot_general(
            xg_ref[0], w_ref[...],
            (((1,), (0,)), ((), ())),
            preferred_element_type=jnp.float32,
        )

        for h in range(N_DEV - 1):
            pltpu.make_async_remote_copy(
                src_ref=wbuf_ref.at[h],
                dst_ref=wbuf_ref.at[h],
                send_sem=w_send.at[h],
                recv_sem=w_recv.at[h],
                device_id=(left,),
                device_id_type=pl.DeviceIdType.MESH,
            ).wait_recv()

            if h + 1 < N_DEV - 1:
                r = pltpu.make_async_remote_copy(
                    src_ref=wbuf_ref.at[h],
                    dst_ref=wbuf_ref.at[h + 1],
                    send_sem=w_send.at[h + 1],
                    recv_sem=w_recv.at[h + 1],
                    device_id=(right,),
                    device_id_type=pl.DeviceIdType.MESH,
                )
                r.start()
                w_rdmas.append(r)

            pltpu.make_async_remote_copy(
                src_ref=xg_ref.at[h + 1],
                dst_ref=xg_ref.at[h + 1],
                send_sem=x_recv.at[h + 1],
                recv_sem=x_recv.at[h + 1],
                device_id=(left,),
                device_id_type=pl.DeviceIdType.MESH,
            ).wait_recv()

            out_ref[...] = out_ref[...] + lax.dot_general(
                xg_ref[h + 1], wbuf_ref[h],
                (((1,), (0,)), ((), ())),
                preferred_element_type=jnp.float32,
            )

        s = sx_ref[0] * sw_ref[0]
        out_ref[...] = jnp.maximum(out_ref[...] * s, 0.0)

        for r in x_rdmas + w_rdmas:
            r.wait_send()

    return pl.pallas_call(
        body,
        out_shape=jax.ShapeDtypeStruct((B, N), jnp.float32),
        in_specs=[
            pl.BlockSpec(memory_space=pltpu.VMEM),
            pl.BlockSpec(memory_space=pltpu.VMEM),
            pl.BlockSpec(memory_space=pltpu.SMEM),
            pl.BlockSpec(memory_space=pltpu.SMEM),
        ],
        out_specs=pl.BlockSpec(memory_space=pltpu.VMEM),
        scratch_shapes=[
            pltpu.VMEM((N_DEV, B, K_shard), jnp.float8_e5m2),
            pltpu.VMEM((N_DEV - 1, K_shard, N), jnp.float8_e5m2),
            pltpu.SemaphoreType.DMA((N_DEV,)),
            pltpu.SemaphoreType.DMA((N_DEV,)),
            pltpu.SemaphoreType.DMA((N_DEV - 1,)),
            pltpu.SemaphoreType.DMA((N_DEV - 1,)),
        ],
        compiler_params=pltpu.CompilerParams(
            collective_id=0,
            vmem_limit_bytes=100 * 1024 * 1024,
        ),
    )(x8, w8, scale_x, scale_w)


# device time: 240152 ns/iter; 1.5909x vs baseline; 1.5909x over previous
import jax
import jax.numpy as jnp
from jax import lax
from jax.experimental import pallas as pl
from jax.experimental.pallas import tpu as pltpu

N_DEV = 8


def kernel(x, w_mat, scale_x, scale_w):
    M, K_shard = x.shape
    _, N = w_mat.shape
    B = M // N_DEV

    x8 = x.astype(jnp.float8_e5m2)
    w8 = w_mat.astype(jnp.float8_e5m2)

    R_HOPS = 4
    L_HOPS = 3

    def body(x_ref, w_ref, sx_ref, sw_ref, out_ref,
             xg_ref, rbuf_ref, lbuf_ref, x_send, x_recv,
             r_send, r_recv, l_send, l_recv):
        me = lax.axis_index("i")
        right = lax.rem(me + 1, N_DEV)
        left = lax.rem(me + N_DEV - 1, N_DEV)

        bar = pltpu.get_barrier_semaphore()
        for off in range(1, N_DEV):
            peer = lax.rem(me + off, N_DEV)
            pl.semaphore_signal(bar, inc=1, device_id=(peer,),
                                device_id_type=pl.DeviceIdType.MESH)
        pl.semaphore_wait(bar, N_DEV - 1)

        x_rdmas = []
        for off in range(1, N_DEV):
            dst = lax.rem(me + off, N_DEV)
            r = pltpu.make_async_remote_copy(
                src_ref=x_ref.at[pl.ds(dst * B, B), :],
                dst_ref=xg_ref.at[off],
                send_sem=x_send.at[off],
                recv_sem=x_recv.at[off],
                device_id=(dst,),
                device_id_type=pl.DeviceIdType.MESH,
            )
            r.start()
            x_rdmas.append(r)

        xg_ref[0] = x_ref[pl.ds(me * B, B), :]

        w_rdmas = []
        for buf, sends, recvs, nbr in (
            (rbuf_ref, r_send, r_recv, right),
            (lbuf_ref, l_send, l_recv, left),
        ):
            r0 = pltpu.make_async_remote_copy(
                src_ref=w_ref,
                dst_ref=buf.at[0],
                send_sem=sends.at[0],
                recv_sem=recvs.at[0],
                device_id=(nbr,),
                device_id_type=pl.DeviceIdType.MESH,
            )
            r0.start()
            w_rdmas.append(r0)

        out_ref[...] = lax.dot_general(
            xg_ref[0], w_ref[...],
            (((1,), (0,)), ((), ())),
            preferred_element_type=jnp.float32,
        )

        seq = [("r", 0), ("l", 0), ("r", 1), ("l", 1), ("r", 2), ("l", 2),
               ("r", 3)]
        for dirn, k in seq:
            if dirn == "r":
                buf, sends, recvs = rbuf_ref, r_send, r_recv
                src_nbr, fwd_nbr, hops, slot = left, right, R_HOPS, k + 1
            else:
                buf, sends, recvs = lbuf_ref, l_send, l_recv
                src_nbr, fwd_nbr, hops, slot = right, left, L_HOPS, 7 - k
            pltpu.make_async_remote_copy(
                src_ref=buf.at[k],
                dst_ref=buf.at[k],
                send_sem=sends.at[k],
                recv_sem=recvs.at[k],
                device_id=(src_nbr,),
                device_id_type=pl.DeviceIdType.MESH,
            ).wait_recv()

            if k + 1 < hops:
                r = pltpu.make_async_remote_copy(
                    src_ref=buf.at[k],
                    dst_ref=buf.at[k + 1],
                    send_sem=sends.at[k + 1],
                    recv_sem=recvs.at[k + 1],
                    device_id=(fwd_nbr,),
                    device_id_type=pl.DeviceIdType.MESH,
                )
                r.start()
                w_rdmas.append(r)

            pltpu.make_async_remote_copy(
                src_ref=xg_ref.at[slot],
                dst_ref=xg_ref.at[slot],
                send_sem=x_recv.at[slot],
                recv_sem=x_recv.at[slot],
                device_id=(src_nbr,),
                device_id_type=pl.DeviceIdType.MESH,
            ).wait_recv()

            out_ref[...] = out_ref[...] + lax.dot_general(
                xg_ref[slot], buf[k],
                (((1,), (0,)), ((), ())),
                preferred_element_type=jnp.float32,
            )

        s = sx_ref[0] * sw_ref[0]
        out_ref[...] = jnp.maximum(out_ref[...] * s, 0.0)

        for r in x_rdmas + w_rdmas:
            r.wait_send()

    return pl.pallas_call(
        body,
        out_shape=jax.ShapeDtypeStruct((B, N), jnp.float32),
        in_specs=[
            pl.BlockSpec(memory_space=pltpu.VMEM),
            pl.BlockSpec(memory_space=pltpu.VMEM),
            pl.BlockSpec(memory_space=pltpu.SMEM),
            pl.BlockSpec(memory_space=pltpu.SMEM),
        ],
        out_specs=pl.BlockSpec(memory_space=pltpu.VMEM),
        scratch_shapes=[
            pltpu.VMEM((N_DEV, B, K_shard), jnp.float8_e5m2),
            pltpu.VMEM((R_HOPS, K_shard, N), jnp.float8_e5m2),
            pltpu.VMEM((L_HOPS, K_shard, N), jnp.float8_e5m2),
            pltpu.SemaphoreType.DMA((N_DEV,)),
            pltpu.SemaphoreType.DMA((N_DEV,)),
            pltpu.SemaphoreType.DMA((R_HOPS,)),
            pltpu.SemaphoreType.DMA((R_HOPS,)),
            pltpu.SemaphoreType.DMA((L_HOPS,)),
            pltpu.SemaphoreType.DMA((L_HOPS,)),
        ],
        compiler_params=pltpu.CompilerParams(
            collective_id=0,
            vmem_limit_bytes=100 * 1024 * 1024,
        ),
    )(x8, w8, scale_x, scale_w)


# device time: 222681 ns/iter; 1.7158x vs baseline; 1.0785x over previous
import jax
import jax.numpy as jnp
from jax import lax
from jax.experimental import pallas as pl
from jax.experimental.pallas import tpu as pltpu

N_DEV = 8


def kernel(x, w_mat, scale_x, scale_w):
    M, K_shard = x.shape
    _, N = w_mat.shape
    B = M // N_DEV

    x8 = x.astype(jnp.float8_e5m2)
    w8 = w_mat.astype(jnp.float8_e5m2)

    R_HOPS = 4
    L_HOPS = 4

    def body(x_ref, w_ref, sx_ref, sw_ref, out_ref,
             xg_ref, rbuf_ref, lbuf_ref, x_send, x_recv,
             r_send, r_recv, l_send, l_recv):
        me = lax.axis_index("i")
        right = lax.rem(me + 1, N_DEV)
        left = lax.rem(me + N_DEV - 1, N_DEV)

        bar = pltpu.get_barrier_semaphore()
        for off in range(1, N_DEV):
            peer = lax.rem(me + off, N_DEV)
            pl.semaphore_signal(bar, inc=1, device_id=(peer,),
                                device_id_type=pl.DeviceIdType.MESH)
        pl.semaphore_wait(bar, N_DEV - 1)

        x_rdmas = []
        for off in (1, 7, 2, 6, 3, 5, 4):
            dst = lax.rem(me + off, N_DEV)
            r = pltpu.make_async_remote_copy(
                src_ref=x_ref.at[pl.ds(dst * B, B), :],
                dst_ref=xg_ref.at[off],
                send_sem=x_send.at[off],
                recv_sem=x_recv.at[off],
                device_id=(dst,),
                device_id_type=pl.DeviceIdType.MESH,
            )
            r.start()
            x_rdmas.append(r)

        xg_ref[0] = x_ref[pl.ds(me * B, B), :]

        w_rdmas = []
        for buf, sends, recvs, nbr in (
            (rbuf_ref, r_send, r_recv, right),
            (lbuf_ref, l_send, l_recv, left),
        ):
            r0 = pltpu.make_async_remote_copy(
                src_ref=w_ref,
                dst_ref=buf.at[0],
                send_sem=sends.at[0],
                recv_sem=recvs.at[0],
                device_id=(nbr,),
                device_id_type=pl.DeviceIdType.MESH,
            )
            r0.start()
            w_rdmas.append(r0)

        out_ref[...] = lax.dot_general(
            xg_ref[0], w_ref[...],
            (((1,), (0,)), ((), ())),
            preferred_element_type=jnp.float32,
        )

        HALF = K_shard // 2
        seq = [("r", 0), ("l", 0), ("r", 1), ("l", 1), ("r", 2), ("l", 2),
               ("r", 3), ("l", 3)]
        for dirn, k in seq:
            if dirn == "r":
                buf, sends, recvs = rbuf_ref, r_send, r_recv
                src_nbr, fwd_nbr = left, right
                slot = k + 1
                row0 = 0
            else:
                buf, sends, recvs = lbuf_ref, l_send, l_recv
                src_nbr, fwd_nbr = right, left
                slot = 7 - k if k < 3 else 4
                row0 = HALF
            is_half = k == 3
            dst_part = (
                buf.at[k, pl.ds(row0, HALF), :] if is_half else buf.at[k]
            )
            pltpu.make_async_remote_copy(
                src_ref=dst_part,
                dst_ref=dst_part,
                send_sem=sends.at[k],
                recv_sem=recvs.at[k],
                device_id=(src_nbr,),
                device_id_type=pl.DeviceIdType.MESH,
            ).wait_recv()

            if k < 2:
                r = pltpu.make_async_remote_copy(
                    src_ref=buf.at[k],
                    dst_ref=buf.at[k + 1],
                    send_sem=sends.at[k + 1],
                    recv_sem=recvs.at[k + 1],
                    device_id=(fwd_nbr,),
                    device_id_type=pl.DeviceIdType.MESH,
                )
                r.start()
                w_rdmas.append(r)
            elif k == 2:
                r = pltpu.make_async_remote_copy(
                    src_ref=buf.at[k, pl.ds(row0, HALF), :],
                    dst_ref=buf.at[k + 1, pl.ds(row0, HALF), :],
                    send_sem=sends.at[k + 1],
                    recv_sem=recvs.at[k + 1],
                    device_id=(fwd_nbr,),
                    device_id_type=pl.DeviceIdType.MESH,
                )
                r.start()
                w_rdmas.append(r)

            if not (dirn == "l" and k == 3):
                pltpu.make_async_remote_copy(
                    src_ref=xg_ref.at[slot],
                    dst_ref=xg_ref.at[slot],
                    send_sem=x_recv.at[slot],
                    recv_sem=x_recv.at[slot],
                    device_id=(src_nbr,),
                    device_id_type=pl.DeviceIdType.MESH,
                ).wait_recv()

            if is_half:
                a = xg_ref[slot, :, row0:row0 + HALF]
                w_chunk = buf[k, row0:row0 + HALF, :]
            else:
                a = xg_ref[slot]
                w_chunk = buf[k]
            out_ref[...] = out_ref[...] + lax.dot_general(
                a, w_chunk,
                (((1,), (0,)), ((), ())),
                preferred_element_type=jnp.float32,
            )

        s = sx_ref[0] * sw_ref[0]
        out_ref[...] = jnp.maximum(out_ref[...] * s, 0.0)

        for r in x_rdmas + w_rdmas:
            r.wait_send()

    return pl.pallas_call(
        body,
        out_shape=jax.ShapeDtypeStruct((B, N), jnp.float32),
        in_specs=[
            pl.BlockSpec(memory_space=pltpu.VMEM),
            pl.BlockSpec(memory_space=pltpu.VMEM),
            pl.BlockSpec(memory_space=pltpu.SMEM),
            pl.BlockSpec(memory_space=pltpu.SMEM),
        ],
        out_specs=pl.BlockSpec(memory_space=pltpu.VMEM),
        scratch_shapes=[
            pltpu.VMEM((N_DEV, B, K_shard), jnp.float8_e5m2),
            pltpu.VMEM((R_HOPS, K_shard, N), jnp.float8_e5m2),
            pltpu.VMEM((L_HOPS, K_shard, N), jnp.float8_e5m2),
            pltpu.SemaphoreType.DMA((N_DEV,)),
            pltpu.SemaphoreType.DMA((N_DEV,)),
            pltpu.SemaphoreType.DMA((R_HOPS,)),
            pltpu.SemaphoreType.DMA((R_HOPS,)),
            pltpu.SemaphoreType.DMA((L_HOPS,)),
            pltpu.SemaphoreType.DMA((L_HOPS,)),
        ],
        compiler_params=pltpu.CompilerParams(
            collective_id=0,
            vmem_limit_bytes=100 * 1024 * 1024,
        ),
    )(x8, w8, scale_x, scale_w)


# device time: 220459 ns/iter; 1.7330x vs baseline; 1.0101x over previous
import jax
import jax.numpy as jnp
from jax import lax
from jax.experimental import pallas as pl
from jax.experimental.pallas import tpu as pltpu

N_DEV = 8


def kernel(x, w_mat, scale_x, scale_w):
    M, K_shard = x.shape
    _, N = w_mat.shape
    B = M // N_DEV

    x8 = x.astype(jnp.float8_e5m2)
    w8 = w_mat.astype(jnp.float8_e5m2)

    R_HOPS = 4
    L_HOPS = 4

    def body(x_ref, w_ref, sx_ref, sw_ref, out_ref,
             xg_ref, rbuf_ref, lbuf_ref, x_send, x_recv,
             r_send, r_recv, l_send, l_recv):
        me = lax.axis_index("i")

        def ring_id(p):
            return jnp.where(p < 4, p, 11 - p)

        pos = ring_id(me)
        right = ring_id(lax.rem(pos + 1, N_DEV))
        left = ring_id(lax.rem(pos + N_DEV - 1, N_DEV))

        bar = pltpu.get_barrier_semaphore()
        for off in range(1, N_DEV):
            peer = lax.rem(me + off, N_DEV)
            pl.semaphore_signal(bar, inc=1, device_id=(peer,),
                                device_id_type=pl.DeviceIdType.MESH)
        pl.semaphore_wait(bar, N_DEV - 1)

        x_rdmas = []
        for off in (1, 7, 2, 6, 3, 5, 4):
            dst = ring_id(lax.rem(pos + off, N_DEV))
            r = pltpu.make_async_remote_copy(
                src_ref=x_ref.at[pl.ds(dst * B, B), :],
                dst_ref=xg_ref.at[off],
                send_sem=x_send.at[off],
                recv_sem=x_recv.at[off],
                device_id=(dst,),
                device_id_type=pl.DeviceIdType.MESH,
            )
            r.start()
            x_rdmas.append(r)

        xg_ref[0] = x_ref[pl.ds(me * B, B), :]

        w_rdmas = []
        for buf, sends, recvs, nbr in (
            (rbuf_ref, r_send, r_recv, right),
            (lbuf_ref, l_send, l_recv, left),
        ):
            r0 = pltpu.make_async_remote_copy(
                src_ref=w_ref,
                dst_ref=buf.at[0],
                send_sem=sends.at[0],
                recv_sem=recvs.at[0],
                device_id=(nbr,),
                device_id_type=pl.DeviceIdType.MESH,
            )
            r0.start()
            w_rdmas.append(r0)

        out_ref[...] = lax.dot_general(
            xg_ref[0], w_ref[...],
            (((1,), (0,)), ((), ())),
            preferred_element_type=jnp.float32,
        )

        HALF = K_shard // 2
        seq = [("r", 0), ("l", 0), ("r", 1), ("l", 1), ("r", 2), ("l", 2),
               ("r", 3), ("l", 3)]
        for dirn, k in seq:
            if dirn == "r":
                buf, sends, recvs = rbuf_ref, r_send, r_recv
                src_nbr, fwd_nbr = left, right
                slot = k + 1
                row0 = 0
            else:
                buf, sends, recvs = lbuf_ref, l_send, l_recv
                src_nbr, fwd_nbr = right, left
                slot = 7 - k if k < 3 else 4
                row0 = HALF
            is_half = k == 3
            dst_part = (
                buf.at[k, pl.ds(row0, HALF), :] if is_half else buf.at[k]
            )
            pltpu.make_async_remote_copy(
                src_ref=dst_part,
                dst_ref=dst_part,
                send_sem=sends.at[k],
                recv_sem=recvs.at[k],
                device_id=(src_nbr,),
                device_id_type=pl.DeviceIdType.MESH,
            ).wait_recv()

            if k < 2:
                r = pltpu.make_async_remote_copy(
                    src_ref=buf.at[k],
                    dst_ref=buf.at[k + 1],
                    send_sem=sends.at[k + 1],
                    recv_sem=recvs.at[k + 1],
                    device_id=(fwd_nbr,),
                    device_id_type=pl.DeviceIdType.MESH,
                )
                r.start()
                w_rdmas.append(r)
            elif k == 2:
                r = pltpu.make_async_remote_copy(
                    src_ref=buf.at[k, pl.ds(row0, HALF), :],
                    dst_ref=buf.at[k + 1, pl.ds(row0, HALF), :],
                    send_sem=sends.at[k + 1],
                    recv_sem=recvs.at[k + 1],
                    device_id=(fwd_nbr,),
                    device_id_type=pl.DeviceIdType.MESH,
                )
                r.start()
                w_rdmas.append(r)

            if not (dirn == "l" and k == 3):
                pltpu.make_async_remote_copy(
                    src_ref=xg_ref.at[slot],
                    dst_ref=xg_ref.at[slot],
                    send_sem=x_recv.at[slot],
                    recv_sem=x_recv.at[slot],
                    device_id=(src_nbr,),
                    device_id_type=pl.DeviceIdType.MESH,
                ).wait_recv()

            if is_half:
                a = xg_ref[slot, :, row0:row0 + HALF]
                w_chunk = buf[k, row0:row0 + HALF, :]
            else:
                a = xg_ref[slot]
                w_chunk = buf[k]
            acc = out_ref[...] + lax.dot_general(
                a, w_chunk,
                (((1,), (0,)), ((), ())),
                preferred_element_type=jnp.float32,
            )
            if (dirn, k) == seq[-1]:
                s = sx_ref[0] * sw_ref[0]
                acc = jnp.maximum(acc * s, 0.0)
            out_ref[...] = acc

        for r in x_rdmas + w_rdmas:
            r.wait_send()

    return pl.pallas_call(
        body,
        out_shape=jax.ShapeDtypeStruct((B, N), jnp.float32),
        in_specs=[
            pl.BlockSpec(memory_space=pltpu.VMEM),
            pl.BlockSpec(memory_space=pltpu.VMEM),
            pl.BlockSpec(memory_space=pltpu.SMEM),
            pl.BlockSpec(memory_space=pltpu.SMEM),
        ],
        out_specs=pl.BlockSpec(memory_space=pltpu.VMEM),
        scratch_shapes=[
            pltpu.VMEM((N_DEV, B, K_shard), jnp.float8_e5m2),
            pltpu.VMEM((R_HOPS, K_shard, N), jnp.float8_e5m2),
            pltpu.VMEM((L_HOPS, K_shard, N), jnp.float8_e5m2),
            pltpu.SemaphoreType.DMA((N_DEV,)),
            pltpu.SemaphoreType.DMA((N_DEV,)),
            pltpu.SemaphoreType.DMA((R_HOPS,)),
            pltpu.SemaphoreType.DMA((R_HOPS,)),
            pltpu.SemaphoreType.DMA((L_HOPS,)),
            pltpu.SemaphoreType.DMA((L_HOPS,)),
        ],
        compiler_params=pltpu.CompilerParams(
            collective_id=0,
            vmem_limit_bytes=100 * 1024 * 1024,
        ),
    )(x8, w8, scale_x, scale_w)


# device time: 172433 ns/iter; 2.2157x vs baseline; 1.2785x over previous
import jax
import jax.numpy as jnp
from jax import lax
from jax.experimental import pallas as pl
from jax.experimental.pallas import tpu as pltpu

N_DEV = 8


TREES = (
    {0: (1, 2, 4), 2: (3,), 4: (5, 6), 3: (7,)},
    {0: (1, 2, 4), 1: (3, 5), 4: (6,), 6: (7,)},
    {0: (1, 2, 4), 1: (5,), 2: (3, 6), 5: (7,)},
)
U_ORDER = (1, 2, 4, 3, 5, 6, 7)
N_SPLITS = (2816, 2688, 2688)


def _vert(i):
    low = i & 3
    return (i & 4) | (low ^ (low >> 1))


def kernel(x, w_mat, scale_x, scale_w):
    M, K_shard = x.shape
    _, N = w_mat.shape
    B = M // N_DEV
    col0 = (0, N_SPLITS[0], N_SPLITS[0] + N_SPLITS[1])

    x8 = x.astype(jnp.float8_e5m2)
    w8 = w_mat.astype(jnp.float8_e5m2)

    def body(x_ref, w_ref, sx_ref, sw_ref, out_ref,
             xg_ref, wb0_ref, wb1_ref, wb2_ref,
             x_send, x_recv, w_send0, w_recv0, w_send1, w_recv1,
             w_send2, w_recv2):
        wbufs = (wb0_ref, wb1_ref, wb2_ref)
        w_sends = (w_send0, w_send1, w_send2)
        w_recvs = (w_recv0, w_recv1, w_recv2)

        me = lax.axis_index("i")
        v_me = _vert(me)

        def nbr_id(u_mask):
            w = v_me ^ u_mask
            low = w & 3
            return (w & 4) | (low ^ (low >> 1))

        bar = pltpu.get_barrier_semaphore()
        for off in range(1, N_DEV):
            peer = lax.rem(me + off, N_DEV)
            pl.semaphore_signal(bar, inc=1, device_id=(peer,),
                                device_id_type=pl.DeviceIdType.MESH)
        pl.semaphore_wait(bar, N_DEV - 1)

        rdmas = []

        for u in U_ORDER:
            dst = nbr_id(u)
            r = pltpu.make_async_remote_copy(
                src_ref=x_ref.at[pl.ds(dst * B, B), :],
                dst_ref=xg_ref.at[u],
                send_sem=x_send.at[u],
                recv_sem=x_recv.at[u],
                device_id=(dst,),
                device_id_type=pl.DeviceIdType.MESH,
            )
            r.start()
            rdmas.append(r)

        for t in range(3):
            c0, cn = col0[t], N_SPLITS[t]
            for c in TREES[t][0]:
                r = pltpu.make_async_remote_copy(
                    src_ref=w_ref.at[:, pl.ds(c0, cn)],
                    dst_ref=wbufs[t].at[c],
                    send_sem=w_sends[t].at[c],
                    recv_sem=w_recvs[t].at[c],
                    device_id=(nbr_id(c),),
                    device_id_type=pl.DeviceIdType.MESH,
                )
                r.start()
                rdmas.append(r)

        xg_ref[0] = x_ref[pl.ds(me * B, B), :]
        out_ref[...] = lax.dot_general(
            xg_ref[0], w_ref[...],
            (((1,), (0,)), ((), ())),
            preferred_element_type=jnp.float32,
        )

        s = sx_ref[0] * sw_ref[0]

        for u in U_ORDER:
            for t in range(3):
                buf, sends, recvs = wbufs[t], w_sends[t], w_recvs[t]
                pltpu.make_async_remote_copy(
                    src_ref=buf.at[u],
                    dst_ref=buf.at[u],
                    send_sem=sends.at[u],
                    recv_sem=recvs.at[u],
                    device_id=(me,),
                    device_id_type=pl.DeviceIdType.MESH,
                ).wait_recv()

                for c in TREES[t].get(u, ()):
                    r = pltpu.make_async_remote_copy(
                        src_ref=buf.at[u],
                        dst_ref=buf.at[c],
                        send_sem=sends.at[c],
                        recv_sem=recvs.at[c],
                        device_id=(nbr_id(u ^ c),),
                        device_id_type=pl.DeviceIdType.MESH,
                    )
                    r.start()
                    rdmas.append(r)

                if t == 0:
                    pltpu.make_async_remote_copy(
                        src_ref=xg_ref.at[u],
                        dst_ref=xg_ref.at[u],
                        send_sem=x_recv.at[u],
                        recv_sem=x_recv.at[u],
                        device_id=(me,),
                        device_id_type=pl.DeviceIdType.MESH,
                    ).wait_recv()

                c0, cn = col0[t], N_SPLITS[t]
                acc = out_ref[:, c0:c0 + cn] + lax.dot_general(
                    xg_ref[u], buf[u],
                    (((1,), (0,)), ((), ())),
                    preferred_element_type=jnp.float32,
                )
                if u == U_ORDER[-1]:
                    acc = jnp.maximum(acc * s, 0.0)
                out_ref[:, c0:c0 + cn] = acc

        for r in rdmas:
            r.wait_send()

    return pl.pallas_call(
        body,
        out_shape=jax.ShapeDtypeStruct((B, N), jnp.float32),
        in_specs=[
            pl.BlockSpec(memory_space=pltpu.VMEM),
            pl.BlockSpec(memory_space=pltpu.VMEM),
            pl.BlockSpec(memory_space=pltpu.SMEM),
            pl.BlockSpec(memory_space=pltpu.SMEM),
        ],
        out_specs=pl.BlockSpec(memory_space=pltpu.VMEM),
        scratch_shapes=[
            pltpu.VMEM((N_DEV, B, K_shard), jnp.float8_e5m2),
            pltpu.VMEM((N_DEV, K_shard, N_SPLITS[0]), jnp.float8_e5m2),
            pltpu.VMEM((N_DEV, K_shard, N_SPLITS[1]), jnp.float8_e5m2),
            pltpu.VMEM((N_DEV, K_shard, N_SPLITS[2]), jnp.float8_e5m2),
            pltpu.SemaphoreType.DMA((N_DEV,)),
            pltpu.SemaphoreType.DMA((N_DEV,)),
            pltpu.SemaphoreType.DMA((N_DEV,)),
            pltpu.SemaphoreType.DMA((N_DEV,)),
            pltpu.SemaphoreType.DMA((N_DEV,)),
            pltpu.SemaphoreType.DMA((N_DEV,)),
            pltpu.SemaphoreType.DMA((N_DEV,)),
            pltpu.SemaphoreType.DMA((N_DEV,)),
        ],
        compiler_params=pltpu.CompilerParams(
            collective_id=0,
            vmem_limit_bytes=100 * 1024 * 1024,
        ),
    )(x8, w8, scale_x, scale_w)


# device time: 163169 ns/iter; 2.3415x vs baseline; 1.0568x over previous
import jax
import jax.numpy as jnp
from jax import lax
from jax.experimental import pallas as pl
from jax.experimental.pallas import tpu as pltpu

N_DEV = 8


TREES = (
    {0: (1, 2, 4), 2: (3,), 4: (5, 6), 3: (7,)},
    {0: (1, 2, 4), 1: (3, 5), 4: (6,), 6: (7,)},
    {0: (1, 2, 4), 1: (5,), 2: (3, 6), 5: (7,)},
)
U_ORDER = (1, 2, 4, 3, 5, 6, 7)
N_SPLITS = (2816, 2688, 2688)


def _vert(i):
    low = i & 3
    return (i & 4) | (low ^ (low >> 1))


def kernel(x, w_mat, scale_x, scale_w):
    M, K_shard = x.shape
    _, N = w_mat.shape
    B = M // N_DEV
    col0 = (0, N_SPLITS[0], N_SPLITS[0] + N_SPLITS[1])

    x8 = x.astype(jnp.float8_e5m2)

    def body(x_ref, w_ref, sx_ref, sw_ref, out_ref,
             xg_ref, wstage_ref, w8_ref, wb0_ref, wb1_ref, wb2_ref,
             dma_sem, x_send, x_recv, w_send0, w_recv0, w_send1, w_recv1,
             w_send2, w_recv2):
        wbufs = (wb0_ref, wb1_ref, wb2_ref)
        w_sends = (w_send0, w_send1, w_send2)
        w_recvs = (w_recv0, w_recv1, w_recv2)

        me = lax.axis_index("i")
        v_me = _vert(me)

        def nbr_id(u_mask):
            w = v_me ^ u_mask
            low = w & 3
            return (w & 4) | (low ^ (low >> 1))

        def stage_dma(t):
            c0, cn = col0[t], N_SPLITS[t]
            return pltpu.make_async_copy(
                w_ref.at[:, pl.ds(c0, cn)],
                wstage_ref.at[:, pl.ds(0, cn)],
                dma_sem,
            )

        stage_dma(0).start()

        bar = pltpu.get_barrier_semaphore()
        for off in range(1, N_DEV):
            peer = lax.rem(me + off, N_DEV)
            pl.semaphore_signal(bar, inc=1, device_id=(peer,),
                                device_id_type=pl.DeviceIdType.MESH)
        pl.semaphore_wait(bar, N_DEV - 1)

        rdmas = []

        for t in range(3):
            c0, cn = col0[t], N_SPLITS[t]
            stage_dma(t).wait()
            w8_ref[:, c0:c0 + cn] = wstage_ref[:, :cn].astype(
                jnp.float8_e5m2)
            if t < 2:
                stage_dma(t + 1).start()
            for c in TREES[t][0]:
                r = pltpu.make_async_remote_copy(
                    src_ref=w8_ref.at[:, pl.ds(c0, cn)],
                    dst_ref=wbufs[t].at[c - 1],
                    send_sem=w_sends[t].at[c],
                    recv_sem=w_recvs[t].at[c],
                    device_id=(nbr_id(c),),
                    device_id_type=pl.DeviceIdType.MESH,
                )
                r.start()
                rdmas.append(r)
            if t == 0:
                for u in U_ORDER:
                    dst = nbr_id(u)
                    r = pltpu.make_async_remote_copy(
                        src_ref=x_ref.at[pl.ds(dst * B, B), :],
                        dst_ref=xg_ref.at[u],
                        send_sem=x_send.at[u],
                        recv_sem=x_recv.at[u],
                        device_id=(dst,),
                        device_id_type=pl.DeviceIdType.MESH,
                    )
                    r.start()
                    rdmas.append(r)

        xg_ref[0] = x_ref[pl.ds(me * B, B), :]
        out_ref[...] = lax.dot_general(
            xg_ref[0], w8_ref[...],
            (((1,), (0,)), ((), ())),
            preferred_element_type=jnp.float32,
        )

        s = sx_ref[0] * sw_ref[0]

        for u in U_ORDER:
            for t in range(3):
                buf, sends, recvs = wbufs[t], w_sends[t], w_recvs[t]
                pltpu.make_async_remote_copy(
                    src_ref=buf.at[u - 1],
                    dst_ref=buf.at[u - 1],
                    send_sem=sends.at[u],
                    recv_sem=recvs.at[u],
                    device_id=(me,),
                    device_id_type=pl.DeviceIdType.MESH,
                ).wait_recv()

                for c in TREES[t].get(u, ()):
                    r = pltpu.make_async_remote_copy(
                        src_ref=buf.at[u - 1],
                        dst_ref=buf.at[c - 1],
                        send_sem=sends.at[c],
                        recv_sem=recvs.at[c],
                        device_id=(nbr_id(u ^ c),),
                        device_id_type=pl.DeviceIdType.MESH,
                    )
                    r.start()
                    rdmas.append(r)

                if t == 0:
                    pltpu.make_async_remote_copy(
                        src_ref=xg_ref.at[u],
                        dst_ref=xg_ref.at[u],
                        send_sem=x_recv.at[u],
                        recv_sem=x_recv.at[u],
                        device_id=(me,),
                        device_id_type=pl.DeviceIdType.MESH,
                    ).wait_recv()

                c0, cn = col0[t], N_SPLITS[t]
                acc = out_ref[:, c0:c0 + cn] + lax.dot_general(
                    xg_ref[u], buf[u - 1],
                    (((1,), (0,)), ((), ())),
                    preferred_element_type=jnp.float32,
                )
                if u == U_ORDER[-1]:
                    acc = jnp.maximum(acc * s, 0.0)
                out_ref[:, c0:c0 + cn] = acc

        for r in rdmas:
            r.wait_send()

    return pl.pallas_call(
        body,
        out_shape=jax.ShapeDtypeStruct((B, N), jnp.float32),
        in_specs=[
            pl.BlockSpec(memory_space=pltpu.VMEM),
            pl.BlockSpec(memory_space=pl.ANY),
            pl.BlockSpec(memory_space=pltpu.SMEM),
            pl.BlockSpec(memory_space=pltpu.SMEM),
        ],
        out_specs=pl.BlockSpec(memory_space=pltpu.VMEM),
        scratch_shapes=[
            pltpu.VMEM((N_DEV, B, K_shard), jnp.float8_e5m2),
            pltpu.VMEM((K_shard, N_SPLITS[0]), jnp.float32),
            pltpu.VMEM((K_shard, N), jnp.float8_e5m2),
            pltpu.VMEM((N_DEV - 1, K_shard, N_SPLITS[0]), jnp.float8_e5m2),
            pltpu.VMEM((N_DEV - 1, K_shard, N_SPLITS[1]), jnp.float8_e5m2),
            pltpu.VMEM((N_DEV - 1, K_shard, N_SPLITS[2]), jnp.float8_e5m2),
            pltpu.SemaphoreType.DMA,
            pltpu.SemaphoreType.DMA((N_DEV,)),
            pltpu.SemaphoreType.DMA((N_DEV,)),
            pltpu.SemaphoreType.DMA((N_DEV,)),
            pltpu.SemaphoreType.DMA((N_DEV,)),
            pltpu.SemaphoreType.DMA((N_DEV,)),
            pltpu.SemaphoreType.DMA((N_DEV,)),
            pltpu.SemaphoreType.DMA((N_DEV,)),
            pltpu.SemaphoreType.DMA((N_DEV,)),
        ],
        compiler_params=pltpu.CompilerParams(
            collective_id=0,
            vmem_limit_bytes=63 * 1024 * 1024,
        ),
    )(x8, w_mat, scale_x, scale_w)


# device time: 161850 ns/iter; 2.3606x vs baseline; 1.0081x over previous
import jax
import jax.numpy as jnp
from jax import lax
from jax.experimental import pallas as pl
from jax.experimental.pallas import tpu as pltpu

N_DEV = 8


TREES = (
    {0: (1, 2, 4), 2: (3,), 4: (5, 6), 3: (7,)},
    {0: (1, 2, 4), 1: (3, 5), 4: (6,), 6: (7,)},
    {0: (1, 2, 4), 1: (5,), 2: (3, 6), 5: (7,)},
)
U_ORDER = (1, 2, 4, 3, 5, 6, 7)
N_SPLITS = (2816, 2688, 2688)
HALVES = ((1408, 1408), (1408, 1280), (1408, 1280))


def _vert(i):
    low = i & 3
    return (i & 4) | (low ^ (low >> 1))


def kernel(x, w_mat, scale_x, scale_w):
    M, K_shard = x.shape
    _, N = w_mat.shape
    B = M // N_DEV
    col0 = (0, N_SPLITS[0], N_SPLITS[0] + N_SPLITS[1])

    x8 = x.astype(jnp.float8_e5m2)

    def body(x_ref, w_ref, sx_ref, sw_ref, out_ref,
             xg_ref, wstage_ref, w8_ref, wb0_ref, wb1_ref, wb2_ref,
             dma_sem, x_send, x_recv, w_send0, w_recv0, w_send1, w_recv1,
             w_send2, w_recv2):
        wbufs = (wb0_ref, wb1_ref, wb2_ref)
        w_sends = (w_send0, w_send1, w_send2)
        w_recvs = (w_recv0, w_recv1, w_recv2)

        me = lax.axis_index("i")
        v_me = _vert(me)

        def nbr_id(u_mask):
            w = v_me ^ u_mask
            low = w & 3
            return (w & 4) | (low ^ (low >> 1))

        def stage_dma(t):
            c0, cn = col0[t], N_SPLITS[t]
            return pltpu.make_async_copy(
                w_ref.at[:, pl.ds(c0, cn)],
                wstage_ref.at[:, pl.ds(0, cn)],
                dma_sem,
            )

        stage_dma(0).start()

        bar = pltpu.get_barrier_semaphore()
        for off in range(1, N_DEV):
            peer = lax.rem(me + off, N_DEV)
            pl.semaphore_signal(bar, inc=1, device_id=(peer,),
                                device_id_type=pl.DeviceIdType.MESH)

        stage_dma(0).wait()
        w8_ref[:, 0:N_SPLITS[0]] = wstage_ref[:, :].astype(jnp.float8_e5m2)
        stage_dma(1).start()

        pl.semaphore_wait(bar, N_DEV - 1)

        rdmas = []

        def send_x(us):
            for u in us:
                dst = nbr_id(u)
                r = pltpu.make_async_remote_copy(
                    src_ref=x_ref.at[pl.ds(dst * B, B), :],
                    dst_ref=xg_ref.at[u],
                    send_sem=x_send.at[u],
                    recv_sem=x_recv.at[u],
                    device_id=(dst,),
                    device_id_type=pl.DeviceIdType.MESH,
                )
                r.start()
                rdmas.append(r)

        for t in range(3):
            c0, cn = col0[t], N_SPLITS[t]
            if t > 0:
                stage_dma(t).wait()
                w8_ref[:, c0:c0 + cn] = wstage_ref[:, :cn].astype(
                    jnp.float8_e5m2)
                if t < 2:
                    stage_dma(t + 1).start()
            for c in TREES[t][0]:
                r = pltpu.make_async_remote_copy(
                    src_ref=w8_ref.at[:, pl.ds(c0, cn)],
                    dst_ref=wbufs[t].at[c - 1],
                    send_sem=w_sends[t].at[c],
                    recv_sem=w_recvs[t].at[c],
                    device_id=(nbr_id(c),),
                    device_id_type=pl.DeviceIdType.MESH,
                )
                r.start()
                rdmas.append(r)
            if t == 0:
                send_x((1, 2, 4))
            elif t == 1:
                send_x((3, 5, 6, 7))

        xg_ref[0] = x_ref[pl.ds(me * B, B), :]
        out_ref[...] = lax.dot_general(
            xg_ref[0], w8_ref[...],
            (((1,), (0,)), ((), ())),
            preferred_element_type=jnp.float32,
        )

        s = sx_ref[0] * sw_ref[0]

        def wait_w(t, u, lo, sz, sem_slot):
            buf, sends, recvs = wbufs[t], w_sends[t], w_recvs[t]
            pltpu.make_async_remote_copy(
                src_ref=buf.at[u - 1, :, pl.ds(lo, sz)],
                dst_ref=buf.at[u - 1, :, pl.ds(lo, sz)],
                send_sem=sends.at[sem_slot],
                recv_sem=recvs.at[sem_slot],
                device_id=(me,),
                device_id_type=pl.DeviceIdType.MESH,
            ).wait_recv()

        def wait_x(u):
            pltpu.make_async_remote_copy(
                src_ref=xg_ref.at[u],
                dst_ref=xg_ref.at[u],
                send_sem=x_recv.at[u],
                recv_sem=x_recv.at[u],
                device_id=(me,),
                device_id_type=pl.DeviceIdType.MESH,
            ).wait_recv()

        def gemm(t, u, lo, sz, fin):
            c0 = col0[t]
            acc = out_ref[:, c0 + lo:c0 + lo + sz] + lax.dot_general(
                xg_ref[u], wbufs[t][u - 1, :, lo:lo + sz],
                (((1,), (0,)), ((), ())),
                preferred_element_type=jnp.float32,
            )
            if fin:
                acc = jnp.maximum(acc * s, 0.0)
            out_ref[:, c0 + lo:c0 + lo + sz] = acc

        for u in U_ORDER[:-1]:
            for t in range(3):
                buf, sends, recvs = wbufs[t], w_sends[t], w_recvs[t]
                wait_w(t, u, 0, N_SPLITS[t], u)

                for c in TREES[t].get(u, ()):
                    if c == 7:
                        hA, hB = HALVES[t]
                        for lo, sz, slot in ((0, hA, 7), (hA, hB, 0)):
                            r = pltpu.make_async_remote_copy(
                                src_ref=buf.at[u - 1, :, pl.ds(lo, sz)],
                                dst_ref=buf.at[c - 1, :, pl.ds(lo, sz)],
                                send_sem=sends.at[slot],
                                recv_sem=recvs.at[slot],
                                device_id=(nbr_id(u ^ c),),
                                device_id_type=pl.DeviceIdType.MESH,
                            )
                            r.start()
                            rdmas.append(r)
                    else:
                        r = pltpu.make_async_remote_copy(
                            src_ref=buf.at[u - 1],
                            dst_ref=buf.at[c - 1],
                            send_sem=sends.at[c],
                            recv_sem=recvs.at[c],
                            device_id=(nbr_id(u ^ c),),
                            device_id_type=pl.DeviceIdType.MESH,
                        )
                        r.start()
                        rdmas.append(r)

                if t == 0:
                    wait_x(u)
                gemm(t, u, 0, N_SPLITS[t], fin=False)

        wait_x(7)
        for t in range(3):
            wait_w(t, 7, 0, HALVES[t][0], 7)
            gemm(t, 7, 0, HALVES[t][0], fin=True)
        for t in range(3):
            wait_w(t, 7, HALVES[t][0], HALVES[t][1], 0)
            gemm(t, 7, HALVES[t][0], HALVES[t][1], fin=True)

        for r in rdmas:
            r.wait_send()

    return pl.pallas_call(
        body,
        out_shape=jax.ShapeDtypeStruct((B, N), jnp.float32),
        in_specs=[
            pl.BlockSpec(memory_space=pltpu.VMEM),
            pl.BlockSpec(memory_space=pl.ANY),
            pl.BlockSpec(memory_space=pltpu.SMEM),
            pl.BlockSpec(memory_space=pltpu.SMEM),
        ],
        out_specs=pl.BlockSpec(memory_space=pltpu.VMEM),
        scratch_shapes=[
            pltpu.VMEM((N_DEV, B, K_shard), jnp.float8_e5m2),
            pltpu.VMEM((K_shard, N_SPLITS[0]), jnp.float32),
            pltpu.VMEM((K_shard, N), jnp.float8_e5m2),
            pltpu.VMEM((N_DEV - 1, K_shard, N_SPLITS[0]), jnp.float8_e5m2),
            pltpu.VMEM((N_DEV - 1, K_shard, N_SPLITS[1]), jnp.float8_e5m2),
            pltpu.VMEM((N_DEV - 1, K_shard, N_SPLITS[2]), jnp.float8_e5m2),
            pltpu.SemaphoreType.DMA,
            pltpu.SemaphoreType.DMA((N_DEV,)),
            pltpu.SemaphoreType.DMA((N_DEV,)),
            pltpu.SemaphoreType.DMA((N_DEV,)),
            pltpu.SemaphoreType.DMA((N_DEV,)),
            pltpu.SemaphoreType.DMA((N_DEV,)),
            pltpu.SemaphoreType.DMA((N_DEV,)),
            pltpu.SemaphoreType.DMA((N_DEV,)),
            pltpu.SemaphoreType.DMA((N_DEV,)),
        ],
        compiler_params=pltpu.CompilerParams(
            collective_id=0,
            vmem_limit_bytes=63 * 1024 * 1024,
        ),
    )(x8, w_mat, scale_x, scale_w)
